# baseline (speedup 1.0000x reference)
import functools

import jax
import jax.numpy as jnp
from jax import lax
from jax.experimental import pallas as pl
from jax.experimental.pallas import tpu as pltpu
from jax.experimental.pallas import tpu_sc as plsc

N = 16777216
NC = 2
NS = 16
NW = NC * NS
L = 16
PER_W = N // NW
BLOCK = 16384
NBLK = PER_W // BLOCK
NBINS = 256
EPAD = 272

_mesh = plsc.VectorSubcoreMesh(
    core_axis_name="c", subcore_axis_name="s", num_cores=NC, num_subcores=NS
)
_cparams = pltpu.CompilerParams(needs_layout_passes=False)


def _worker_id():
    return lax.axis_index("s") * NC + lax.axis_index("c")


def _stream_shard(x_hbm, base, bufs, sems, block_body, carry):
    for b in range(2):
        off = pl.multiple_of(base + b * BLOCK, BLOCK)
        pltpu.async_copy(x_hbm.at[pl.ds(off, BLOCK)], bufs[b], sems[b])

    def pair(p, c):
        for b in range(2):
            g = p * 2 + b
            pltpu.make_async_copy(
                x_hbm.at[pl.ds(0, BLOCK)], bufs[b], sems[b]
            ).wait()
            c = block_body(bufs[b], c)

            @pl.when(g + 2 < NBLK)
            def _():
                off = pl.multiple_of(base + (g + 2) * BLOCK, BLOCK)
                pltpu.async_copy(x_hbm.at[pl.ds(off, BLOCK)], bufs[b], sems[b])

        return c

    return lax.fori_loop(0, NBLK // 2, pair, carry)


@functools.partial(
    pl.kernel,
    out_type=jax.ShapeDtypeStruct((NW * 2 * L,), jnp.float32),
    mesh=_mesh,
    compiler_params=_cparams,
    scratch_types=[
        pltpu.VMEM((BLOCK,), jnp.float32),
        pltpu.VMEM((BLOCK,), jnp.float32),
        pltpu.VMEM((2 * L,), jnp.float32),
        pltpu.SemaphoreType.DMA,
        pltpu.SemaphoreType.DMA,
    ],
)
def _minmax(x_hbm, out_hbm, buf0, buf1, res, sem0, sem1):
    wid = _worker_id()
    base = pl.multiple_of(wid * PER_W, PER_W)

    def block_body(bref, c):
        @plsc.parallel_loop(0, BLOCK // L, unroll=8, carry=c)
        def inner(i, c):
            mn, mx = c
            ib = pl.multiple_of(i * L, L)
            v = bref[pl.ds(ib, L)]
            return (jnp.minimum(mn, v), jnp.maximum(mx, v))

        return inner

    init = (
        jnp.full((L,), jnp.inf, jnp.float32),
        jnp.full((L,), -jnp.inf, jnp.float32),
    )
    mn, mx = _stream_shard(x_hbm, base, (buf0, buf1), (sem0, sem1), block_body, init)
    res[pl.ds(0, L)] = mn
    res[pl.ds(L, L)] = mx
    pltpu.sync_copy(res, out_hbm.at[pl.ds(pl.multiple_of(wid * 2 * L, 2 * L), 2 * L)])


@functools.partial(
    pl.kernel,
    out_type=jax.ShapeDtypeStruct((NW * NBINS,), jnp.float32),
    mesh=_mesh,
    compiler_params=_cparams,
    scratch_types=[
        pltpu.VMEM((BLOCK,), jnp.float32),
        pltpu.VMEM((BLOCK,), jnp.float32),
        pltpu.VMEM((EPAD,), jnp.float32),
        pltpu.VMEM((L,), jnp.float32),
        pltpu.VMEM((L * NBINS,), jnp.float32),
        pltpu.VMEM((NBINS,), jnp.float32),
        pltpu.SemaphoreType.DMA,
        pltpu.SemaphoreType.DMA,
    ],
)
def _hist(x_hbm, edges_hi_hbm, params_hbm, out_hbm, buf0, buf1, edges_hi_v, params_v, h2d, h1d, sem0, sem1):
    wid = _worker_id()
    base = pl.multiple_of(wid * PER_W, PER_W)

    pltpu.sync_copy(edges_hi_hbm, edges_hi_v)
    pltpu.sync_copy(params_hbm, params_v)
    pv = params_v[...]
    tmin = pv[0]
    scale = pv[2]

    zeros = jnp.zeros((L,), jnp.float32)

    def zero_it(i, c):
        h2d[pl.ds(pl.multiple_of(i * L, L), L)] = zeros
        return c

    lax.fori_loop(0, L * NBINS // L, zero_it, 0)

    lane = jnp.arange(L, dtype=jnp.int32)
    lane_base = lane * NBINS
    ones = jnp.ones((L,), jnp.float32)
    c255 = jnp.full((L,), 255, jnp.int32)
    czero = jnp.zeros((L,), jnp.int32)

    eps = jnp.float32(0.01)
    fzero = jnp.zeros((L,), jnp.float32)

    def block_body(bref, c):
        @plsc.parallel_loop(0, BLOCK // L, unroll=16)
        def _(i):
            ib = pl.multiple_of(i * L, L)
            v = bref[pl.ds(ib, L)]
            u = jnp.maximum((v - tmin) * scale - eps, fzero)
            idx0 = jnp.minimum(u.astype(jnp.int32), c255)
            e = plsc.load_gather(edges_hi_v, [idx0])
            idx = jnp.where(v >= e, idx0 + 1, idx0)
            idx = jnp.minimum(idx, c255)
            plsc.addupdate_scatter(h2d, [idx + lane_base], ones)

        return c

    _stream_shard(x_hbm, base, (buf0, buf1), (sem0, sem1), block_body, 0)

    def red_it(j, c):
        jb = pl.multiple_of(j * L, L)
        acc = h2d[pl.ds(jb, L)]
        for l in range(1, L):
            acc = acc + h2d[pl.ds(l * NBINS + jb, L)]
        h1d[pl.ds(jb, L)] = acc
        return c

    lax.fori_loop(0, NBINS // L, red_it, 0)
    pltpu.sync_copy(
        h1d, out_hbm.at[pl.ds(pl.multiple_of(wid * NBINS, NBINS), NBINS)]
    )


_W1 = float((1 - 0.99))
_W2 = float((1 + 0.99))


@functools.partial(
    pl.kernel,
    out_type=jax.ShapeDtypeStruct((L,), jnp.float32),
    mesh=_mesh,
    compiler_params=_cparams,
    scratch_types=[
        pltpu.VMEM((NW * NBINS,), jnp.float32),
        pltpu.VMEM((EPAD,), jnp.float32),
        pltpu.VMEM((NBINS,), jnp.float32),
        pltpu.VMEM((L,), jnp.float32),
    ],
)
def _epilogue(parts_hbm, edges_hbm, out_hbm, parts_v, edges_v, cum_v, out_v):
    wid = _worker_id()

    @pl.when(wid == 0)
    def _():
        pltpu.sync_copy(parts_hbm, parts_v)
        pltpu.sync_copy(edges_hbm, edges_v)

        def cum_it(j, carry):
            jb = pl.multiple_of(j * L, L)
            acc = parts_v[pl.ds(jb, L)]
            for w in range(1, NW):
                acc = acc + parts_v[pl.ds(w * NBINS + jb, L)]
            c = plsc.cumsum(acc) + carry
            cum_v[pl.ds(jb, L)] = c
            return jnp.max(c)

        total = lax.fori_loop(0, NBINS // L, cum_it, jnp.float32(0.0))

        t1 = total * jnp.float32(_W1) * jnp.float32(0.5)
        t2 = total * jnp.float32(_W2) * jnp.float32(0.5)

        def cnt_it(j, carry):
            lo, hi = carry
            c = cum_v[pl.ds(pl.multiple_of(j * L, L), L)]
            lo = lo + plsc.all_reduce_population_count(c <= t1)
            hi = hi + plsc.all_reduce_population_count(c <= t2)
            return (lo, hi)

        zi = jnp.zeros((L,), jnp.int32)
        lo_idx, hi_idx = lax.fori_loop(0, NBINS // L, cnt_it, (zi, zi))

        minv = plsc.load_gather(edges_v, [lo_idx])
        maxv = plsc.load_gather(edges_v, [hi_idx])
        lane = jnp.arange(L, dtype=jnp.int32)
        out_v[...] = jnp.where(lane == 0, minv, jnp.where(lane == 1, maxv, 0.0))
        pltpu.sync_copy(out_v, out_hbm)


def kernel(tensor):
    parts = _minmax(tensor).reshape(NW, 2, L)
    tmin = parts[:, 0, :].min()
    tmax = parts[:, 1, :].max()
    edges = jnp.linspace(tmin, tmax, NBINS + 1).astype(jnp.float32)
    scale = jnp.float32(256.0) / (tmax - tmin)
    edges_pad = jnp.concatenate([edges, jnp.zeros((EPAD - (NBINS + 1),), jnp.float32)])
    edges_hi_pad = jnp.concatenate([edges[1:], jnp.zeros((EPAD - NBINS,), jnp.float32)])
    params = jnp.concatenate(
        [jnp.stack([tmin, tmax, scale]), jnp.zeros((L - 3,), jnp.float32)]
    )
    hist_parts = _hist(tensor, edges_hi_pad, params)
    out = _epilogue(hist_parts, edges_pad)
    return (out[0], out[1])


# speedup vs baseline: 1.0707x; 1.0707x over previous
import functools

import jax
import jax.numpy as jnp
from jax import lax
from jax.experimental import pallas as pl
from jax.experimental.pallas import tpu as pltpu
from jax.experimental.pallas import tpu_sc as plsc

N = 16777216
NC = 2
NS = 16
NW = NC * NS
L = 16
PER_W = N // NW
BLOCK = 16384
NBLK = PER_W // BLOCK
NBINS = 256
EPAD = 272

_mesh = plsc.VectorSubcoreMesh(
    core_axis_name="c", subcore_axis_name="s", num_cores=NC, num_subcores=NS
)
_cparams = pltpu.CompilerParams(needs_layout_passes=False)


def _worker_id():
    return lax.axis_index("s") * NC + lax.axis_index("c")


def _stream_shard(x_hbm, base, bufs, sems, block_body, carry):
    for b in range(2):
        off = pl.multiple_of(base + b * BLOCK, BLOCK)
        pltpu.async_copy(x_hbm.at[pl.ds(off, BLOCK)], bufs[b], sems[b])

    def pair(p, c):
        for b in range(2):
            g = p * 2 + b
            pltpu.make_async_copy(
                x_hbm.at[pl.ds(0, BLOCK)], bufs[b], sems[b]
            ).wait()
            c = block_body(bufs[b], c)

            @pl.when(g + 2 < NBLK)
            def _():
                off = pl.multiple_of(base + (g + 2) * BLOCK, BLOCK)
                pltpu.async_copy(x_hbm.at[pl.ds(off, BLOCK)], bufs[b], sems[b])

        return c

    return lax.fori_loop(0, NBLK // 2, pair, carry)


@functools.partial(
    pl.kernel,
    out_type=jax.ShapeDtypeStruct((NW * 2 * L,), jnp.float32),
    mesh=_mesh,
    compiler_params=_cparams,
    scratch_types=[
        pltpu.VMEM((BLOCK,), jnp.float32),
        pltpu.VMEM((BLOCK,), jnp.float32),
        pltpu.VMEM((2 * L,), jnp.float32),
        pltpu.SemaphoreType.DMA,
        pltpu.SemaphoreType.DMA,
    ],
)
def _minmax(x_hbm, out_hbm, buf0, buf1, res, sem0, sem1):
    wid = _worker_id()
    base = pl.multiple_of(wid * PER_W, PER_W)

    def block_body(bref, c):
        @plsc.parallel_loop(0, BLOCK // L, unroll=8, carry=c)
        def inner(i, c):
            mn, mx = c
            ib = pl.multiple_of(i * L, L)
            v = bref[pl.ds(ib, L)]
            return (jnp.minimum(mn, v), jnp.maximum(mx, v))

        return inner

    init = (
        jnp.full((L,), jnp.inf, jnp.float32),
        jnp.full((L,), -jnp.inf, jnp.float32),
    )
    mn, mx = _stream_shard(x_hbm, base, (buf0, buf1), (sem0, sem1), block_body, init)
    res[pl.ds(0, L)] = mn
    res[pl.ds(L, L)] = mx
    pltpu.sync_copy(res, out_hbm.at[pl.ds(pl.multiple_of(wid * 2 * L, 2 * L), 2 * L)])


@functools.partial(
    pl.kernel,
    out_type=jax.ShapeDtypeStruct((NW * NBINS,), jnp.float32),
    mesh=_mesh,
    compiler_params=_cparams,
    scratch_types=[
        pltpu.VMEM((BLOCK,), jnp.float32),
        pltpu.VMEM((BLOCK,), jnp.float32),
        pltpu.VMEM((EPAD,), jnp.float32),
        pltpu.VMEM((L,), jnp.float32),
        pltpu.VMEM((L * NBINS,), jnp.float32),
        pltpu.VMEM((NBINS,), jnp.float32),
        pltpu.SemaphoreType.DMA,
        pltpu.SemaphoreType.DMA,
    ],
)
def _hist(x_hbm, edges_hi_hbm, params_hbm, out_hbm, buf0, buf1, edges_hi_v, params_v, h2d, h1d, sem0, sem1):
    wid = _worker_id()
    base = pl.multiple_of(wid * PER_W, PER_W)

    pltpu.sync_copy(edges_hi_hbm, edges_hi_v)
    pltpu.sync_copy(params_hbm, params_v)
    pv = params_v[...]
    tmin = pv[0]
    scale = pv[2]

    zeros = jnp.zeros((L,), jnp.float32)

    def zero_it(i, c):
        h2d[pl.ds(pl.multiple_of(i * L, L), L)] = zeros
        return c

    lax.fori_loop(0, L * NBINS // L, zero_it, 0)

    lane = jnp.arange(L, dtype=jnp.int32)
    lane_base = lane * NBINS
    ones = jnp.ones((L,), jnp.float32)
    c255 = jnp.full((L,), 255, jnp.int32)
    czero = jnp.zeros((L,), jnp.int32)

    eps = jnp.float32(0.01)
    fzero = jnp.zeros((L,), jnp.float32)

    def block_body(bref, c):
        @plsc.parallel_loop(0, BLOCK // L, unroll=8)
        def _(i):
            ib = pl.multiple_of(i * L, L)
            v = bref[pl.ds(ib, L)]
            u = jnp.maximum((v - tmin) * scale - eps, fzero)
            idx0 = jnp.minimum(u.astype(jnp.int32), c255)
            e = plsc.load_gather(edges_hi_v, [idx0])
            idx = jnp.where(v >= e, idx0 + 1, idx0)
            idx = jnp.minimum(idx, c255)
            plsc.addupdate_scatter(h2d, [idx + lane_base], ones)

        return c

    _stream_shard(x_hbm, base, (buf0, buf1), (sem0, sem1), block_body, 0)

    def red_it(j, c):
        jb = pl.multiple_of(j * L, L)
        acc = h2d[pl.ds(jb, L)]
        for l in range(1, L):
            acc = acc + h2d[pl.ds(l * NBINS + jb, L)]
        h1d[pl.ds(jb, L)] = acc
        return c

    lax.fori_loop(0, NBINS // L, red_it, 0)
    pltpu.sync_copy(
        h1d, out_hbm.at[pl.ds(pl.multiple_of(wid * NBINS, NBINS), NBINS)]
    )


_W1 = float((1 - 0.99))
_W2 = float((1 + 0.99))


@functools.partial(
    pl.kernel,
    out_type=jax.ShapeDtypeStruct((L,), jnp.float32),
    mesh=_mesh,
    compiler_params=_cparams,
    scratch_types=[
        pltpu.VMEM((NW * NBINS,), jnp.float32),
        pltpu.VMEM((EPAD,), jnp.float32),
        pltpu.VMEM((NBINS,), jnp.float32),
        pltpu.VMEM((L,), jnp.float32),
    ],
)
def _epilogue(parts_hbm, edges_hbm, out_hbm, parts_v, edges_v, cum_v, out_v):
    wid = _worker_id()

    @pl.when(wid == 0)
    def _():
        pltpu.sync_copy(parts_hbm, parts_v)
        pltpu.sync_copy(edges_hbm, edges_v)

        def cum_it(j, carry):
            jb = pl.multiple_of(j * L, L)
            acc = parts_v[pl.ds(jb, L)]
            for w in range(1, NW):
                acc = acc + parts_v[pl.ds(w * NBINS + jb, L)]
            c = plsc.cumsum(acc) + carry
            cum_v[pl.ds(jb, L)] = c
            return jnp.max(c)

        total = lax.fori_loop(0, NBINS // L, cum_it, jnp.float32(0.0))

        t1 = total * jnp.float32(_W1) * jnp.float32(0.5)
        t2 = total * jnp.float32(_W2) * jnp.float32(0.5)

        def cnt_it(j, carry):
            lo, hi = carry
            c = cum_v[pl.ds(pl.multiple_of(j * L, L), L)]
            lo = lo + plsc.all_reduce_population_count(c <= t1)
            hi = hi + plsc.all_reduce_population_count(c <= t2)
            return (lo, hi)

        zi = jnp.zeros((L,), jnp.int32)
        lo_idx, hi_idx = lax.fori_loop(0, NBINS // L, cnt_it, (zi, zi))

        minv = plsc.load_gather(edges_v, [lo_idx])
        maxv = plsc.load_gather(edges_v, [hi_idx])
        lane = jnp.arange(L, dtype=jnp.int32)
        out_v[...] = jnp.where(lane == 0, minv, jnp.where(lane == 1, maxv, 0.0))
        pltpu.sync_copy(out_v, out_hbm)


def kernel(tensor):
    parts = _minmax(tensor).reshape(NW, 2, L)
    tmin = parts[:, 0, :].min()
    tmax = parts[:, 1, :].max()
    edges = jnp.linspace(tmin, tmax, NBINS + 1).astype(jnp.float32)
    scale = jnp.float32(256.0) / (tmax - tmin)
    edges_pad = jnp.concatenate([edges, jnp.zeros((EPAD - (NBINS + 1),), jnp.float32)])
    edges_hi_pad = jnp.concatenate([edges[1:], jnp.zeros((EPAD - NBINS,), jnp.float32)])
    params = jnp.concatenate(
        [jnp.stack([tmin, tmax, scale]), jnp.zeros((L - 3,), jnp.float32)]
    )
    hist_parts = _hist(tensor, edges_hi_pad, params)
    out = _epilogue(hist_parts, edges_pad)
    return (out[0], out[1])


# speedup vs baseline: 1.2059x; 1.1262x over previous
import functools

import jax
import jax.numpy as jnp
from jax import lax
from jax.experimental import pallas as pl
from jax.experimental.pallas import tpu as pltpu
from jax.experimental.pallas import tpu_sc as plsc

N = 16777216
NC = 2
NS = 16
NW = NC * NS
L = 16
PER_W = N // NW
BLOCK = 16384
NBLK = PER_W // BLOCK
NBINS = 256
EPAD = 272

_mesh = plsc.VectorSubcoreMesh(
    core_axis_name="c", subcore_axis_name="s", num_cores=NC, num_subcores=NS
)
_cparams = pltpu.CompilerParams(needs_layout_passes=False)


def _worker_id():
    return lax.axis_index("s") * NC + lax.axis_index("c")


def _stream_shard(x_hbm, base, bufs, sems, block_body, carry):
    for b in range(2):
        off = pl.multiple_of(base + b * BLOCK, BLOCK)
        pltpu.async_copy(x_hbm.at[pl.ds(off, BLOCK)], bufs[b], sems[b])

    def pair(p, c):
        for b in range(2):
            g = p * 2 + b
            pltpu.make_async_copy(
                x_hbm.at[pl.ds(0, BLOCK)], bufs[b], sems[b]
            ).wait()
            c = block_body(bufs[b], c)

            @pl.when(g + 2 < NBLK)
            def _():
                off = pl.multiple_of(base + (g + 2) * BLOCK, BLOCK)
                pltpu.async_copy(x_hbm.at[pl.ds(off, BLOCK)], bufs[b], sems[b])

        return c

    return lax.fori_loop(0, NBLK // 2, pair, carry)


@functools.partial(
    pl.kernel,
    out_type=jax.ShapeDtypeStruct((NW * 2 * L,), jnp.float32),
    mesh=_mesh,
    compiler_params=_cparams,
    scratch_types=[
        pltpu.VMEM((BLOCK,), jnp.float32),
        pltpu.VMEM((BLOCK,), jnp.float32),
        pltpu.VMEM((2 * L,), jnp.float32),
        pltpu.SemaphoreType.DMA,
        pltpu.SemaphoreType.DMA,
    ],
)
def _minmax(x_hbm, out_hbm, buf0, buf1, res, sem0, sem1):
    wid = _worker_id()
    base = pl.multiple_of(wid * PER_W, PER_W)

    def block_body(bref, c):
        @plsc.parallel_loop(0, BLOCK // L, unroll=8, carry=c)
        def inner(i, c):
            mn, mx = c
            ib = pl.multiple_of(i * L, L)
            v = bref[pl.ds(ib, L)]
            return (jnp.minimum(mn, v), jnp.maximum(mx, v))

        return inner

    init = (
        jnp.full((L,), jnp.inf, jnp.float32),
        jnp.full((L,), -jnp.inf, jnp.float32),
    )
    mn, mx = _stream_shard(x_hbm, base, (buf0, buf1), (sem0, sem1), block_body, init)
    res[pl.ds(0, L)] = mn
    res[pl.ds(L, L)] = mx
    pltpu.sync_copy(res, out_hbm.at[pl.ds(pl.multiple_of(wid * 2 * L, 2 * L), 2 * L)])


@functools.partial(
    pl.kernel,
    out_type=jax.ShapeDtypeStruct((NW * NBINS,), jnp.float32),
    mesh=_mesh,
    compiler_params=_cparams,
    scratch_types=[
        pltpu.VMEM((BLOCK,), jnp.float32),
        pltpu.VMEM((BLOCK,), jnp.float32),
        pltpu.VMEM((EPAD,), jnp.float32),
        pltpu.VMEM((L,), jnp.float32),
        pltpu.VMEM((L * (NBINS + 1),), jnp.float32),
        pltpu.VMEM((NBINS,), jnp.float32),
        pltpu.SemaphoreType.DMA,
        pltpu.SemaphoreType.DMA,
    ],
)
def _hist(x_hbm, edges_hi_hbm, params_hbm, out_hbm, buf0, buf1, edges_hi_v, params_v, h2d, h1d, sem0, sem1):
    wid = _worker_id()
    base = pl.multiple_of(wid * PER_W, PER_W)
    STRIDE = NBINS + 1

    pltpu.sync_copy(edges_hi_hbm, edges_hi_v)
    pltpu.sync_copy(params_hbm, params_v)
    pv = params_v[...]
    scale = pv[2]
    tmin_eps = pv[3]

    zeros = jnp.zeros((L,), jnp.float32)

    def zero_it(i, c):
        h2d[pl.ds(pl.multiple_of(i * L, L), L)] = zeros
        return c

    lax.fori_loop(0, L * STRIDE // L, zero_it, 0)

    lane = jnp.arange(L, dtype=jnp.int32)
    lane_base = lane * STRIDE
    ones = jnp.ones((L,), jnp.float32)
    fzero = jnp.zeros((L,), jnp.float32)

    def block_body(bref, c):
        @plsc.parallel_loop(0, BLOCK // L, unroll=8)
        def _(i):
            ib = pl.multiple_of(i * L, L)
            v = bref[pl.ds(ib, L)]
            u = jnp.maximum((v - tmin_eps) * scale, fzero)
            idx0 = u.astype(jnp.int32)
            e = plsc.load_gather(edges_hi_v, [idx0])
            idx = idx0 + (v >= e)
            plsc.addupdate_scatter(h2d, [idx + lane_base], ones)

        return c

    _stream_shard(x_hbm, base, (buf0, buf1), (sem0, sem1), block_body, 0)

    def red_it(j, c):
        jb = pl.multiple_of(j * L, L)
        acc = h2d[pl.ds(jb, L)]
        for l in range(1, L):
            acc = acc + h2d[pl.ds(l * STRIDE + jb, L)]
        h1d[pl.ds(jb, L)] = acc
        return c

    lax.fori_loop(0, NBINS // L, red_it, 0)
    t256 = plsc.load_gather(h2d, [lane_base + NBINS])
    s256 = jnp.sum(t256)
    last = h1d[pl.ds(NBINS - L, L)]
    h1d[pl.ds(NBINS - L, L)] = last + jnp.where(lane == L - 1, s256, 0.0)
    pltpu.sync_copy(
        h1d, out_hbm.at[pl.ds(pl.multiple_of(wid * NBINS, NBINS), NBINS)]
    )


_W1 = float((1 - 0.99))
_W2 = float((1 + 0.99))


@functools.partial(
    pl.kernel,
    out_type=jax.ShapeDtypeStruct((L,), jnp.float32),
    mesh=_mesh,
    compiler_params=_cparams,
    scratch_types=[
        pltpu.VMEM((NW * NBINS,), jnp.float32),
        pltpu.VMEM((EPAD,), jnp.float32),
        pltpu.VMEM((NBINS,), jnp.float32),
        pltpu.VMEM((L,), jnp.float32),
    ],
)
def _epilogue(parts_hbm, edges_hbm, out_hbm, parts_v, edges_v, cum_v, out_v):
    wid = _worker_id()

    @pl.when(wid == 0)
    def _():
        pltpu.sync_copy(parts_hbm, parts_v)
        pltpu.sync_copy(edges_hbm, edges_v)

        def cum_it(j, carry):
            jb = pl.multiple_of(j * L, L)
            acc = parts_v[pl.ds(jb, L)]
            for w in range(1, NW):
                acc = acc + parts_v[pl.ds(w * NBINS + jb, L)]
            c = plsc.cumsum(acc) + carry
            cum_v[pl.ds(jb, L)] = c
            return jnp.max(c)

        total = lax.fori_loop(0, NBINS // L, cum_it, jnp.float32(0.0))

        t1 = total * jnp.float32(_W1) * jnp.float32(0.5)
        t2 = total * jnp.float32(_W2) * jnp.float32(0.5)

        def cnt_it(j, carry):
            lo, hi = carry
            c = cum_v[pl.ds(pl.multiple_of(j * L, L), L)]
            lo = lo + plsc.all_reduce_population_count(c <= t1)
            hi = hi + plsc.all_reduce_population_count(c <= t2)
            return (lo, hi)

        zi = jnp.zeros((L,), jnp.int32)
        lo_idx, hi_idx = lax.fori_loop(0, NBINS // L, cnt_it, (zi, zi))

        minv = plsc.load_gather(edges_v, [lo_idx])
        maxv = plsc.load_gather(edges_v, [hi_idx])
        lane = jnp.arange(L, dtype=jnp.int32)
        out_v[...] = jnp.where(lane == 0, minv, jnp.where(lane == 1, maxv, 0.0))
        pltpu.sync_copy(out_v, out_hbm)


def kernel(tensor):
    parts = _minmax(tensor).reshape(NW, 2, L)
    tmin = parts[:, 0, :].min()
    tmax = parts[:, 1, :].max()
    edges = jnp.linspace(tmin, tmax, NBINS + 1).astype(jnp.float32)
    scale = jnp.float32(256.0) / (tmax - tmin)
    edges_pad = jnp.concatenate([edges, jnp.zeros((EPAD - (NBINS + 1),), jnp.float32)])
    edges_hi_pad = jnp.concatenate([edges[1:], jnp.zeros((EPAD - NBINS,), jnp.float32)])
    tmin_eps = tmin + jnp.float32(0.01) / scale
    params = jnp.concatenate(
        [jnp.stack([tmin, tmax, scale, tmin_eps]), jnp.zeros((L - 4,), jnp.float32)]
    )
    hist_parts = _hist(tensor, edges_hi_pad, params)
    out = _epilogue(hist_parts, edges_pad)
    return (out[0], out[1])


# speedup vs baseline: 1.2241x; 1.0151x over previous
import functools

import jax
import jax.numpy as jnp
from jax import lax
from jax.experimental import pallas as pl
from jax.experimental.pallas import tpu as pltpu
from jax.experimental.pallas import tpu_sc as plsc

N = 16777216
NC = 2
NS = 16
NW = NC * NS
L = 16
PER_W = N // NW
BLOCK = 32768
NBLK = PER_W // BLOCK
NBINS = 256
EPAD = 272

_mesh = plsc.VectorSubcoreMesh(
    core_axis_name="c", subcore_axis_name="s", num_cores=NC, num_subcores=NS
)
_cparams = pltpu.CompilerParams(needs_layout_passes=False)


def _worker_id():
    return lax.axis_index("s") * NC + lax.axis_index("c")


def _stream_shard(x_hbm, base, bufs, sems, block_body, carry):
    for b in range(2):
        off = pl.multiple_of(base + b * BLOCK, BLOCK)
        pltpu.async_copy(x_hbm.at[pl.ds(off, BLOCK)], bufs[b], sems[b])

    def pair(p, c):
        for b in range(2):
            g = p * 2 + b
            pltpu.make_async_copy(
                x_hbm.at[pl.ds(0, BLOCK)], bufs[b], sems[b]
            ).wait()
            c = block_body(bufs[b], c)

            @pl.when(g + 2 < NBLK)
            def _():
                off = pl.multiple_of(base + (g + 2) * BLOCK, BLOCK)
                pltpu.async_copy(x_hbm.at[pl.ds(off, BLOCK)], bufs[b], sems[b])

        return c

    return lax.fori_loop(0, NBLK // 2, pair, carry)


@functools.partial(
    pl.kernel,
    out_type=jax.ShapeDtypeStruct((NW * 2 * L,), jnp.float32),
    mesh=_mesh,
    compiler_params=_cparams,
    scratch_types=[
        pltpu.VMEM((BLOCK,), jnp.float32),
        pltpu.VMEM((BLOCK,), jnp.float32),
        pltpu.VMEM((2 * L,), jnp.float32),
        pltpu.SemaphoreType.DMA,
        pltpu.SemaphoreType.DMA,
    ],
)
def _minmax(x_hbm, out_hbm, buf0, buf1, res, sem0, sem1):
    wid = _worker_id()
    base = pl.multiple_of(wid * PER_W, PER_W)

    def block_body(bref, c):
        @plsc.parallel_loop(0, BLOCK // L, unroll=8, carry=c)
        def inner(i, c):
            mn, mx = c
            ib = pl.multiple_of(i * L, L)
            v = bref[pl.ds(ib, L)]
            return (jnp.minimum(mn, v), jnp.maximum(mx, v))

        return inner

    init = (
        jnp.full((L,), jnp.inf, jnp.float32),
        jnp.full((L,), -jnp.inf, jnp.float32),
    )
    mn, mx = _stream_shard(x_hbm, base, (buf0, buf1), (sem0, sem1), block_body, init)
    res[pl.ds(0, L)] = mn
    res[pl.ds(L, L)] = mx
    pltpu.sync_copy(res, out_hbm.at[pl.ds(pl.multiple_of(wid * 2 * L, 2 * L), 2 * L)])


@functools.partial(
    pl.kernel,
    out_type=jax.ShapeDtypeStruct((NW * NBINS,), jnp.float32),
    mesh=_mesh,
    compiler_params=_cparams,
    scratch_types=[
        pltpu.VMEM((BLOCK,), jnp.float32),
        pltpu.VMEM((BLOCK,), jnp.float32),
        pltpu.VMEM((EPAD,), jnp.float32),
        pltpu.VMEM((L,), jnp.float32),
        pltpu.VMEM((L * (NBINS + 1),), jnp.float32),
        pltpu.VMEM((NBINS,), jnp.float32),
        pltpu.SemaphoreType.DMA,
        pltpu.SemaphoreType.DMA,
    ],
)
def _hist(x_hbm, edges_hi_hbm, params_hbm, out_hbm, buf0, buf1, edges_hi_v, params_v, h2d, h1d, sem0, sem1):
    wid = _worker_id()
    base = pl.multiple_of(wid * PER_W, PER_W)
    STRIDE = NBINS + 1

    pltpu.sync_copy(edges_hi_hbm, edges_hi_v)
    pltpu.sync_copy(params_hbm, params_v)
    pv = params_v[...]
    scale = pv[2]
    tmin_eps = pv[3]

    zeros = jnp.zeros((L,), jnp.float32)

    def zero_it(i, c):
        h2d[pl.ds(pl.multiple_of(i * L, L), L)] = zeros
        return c

    lax.fori_loop(0, L * STRIDE // L, zero_it, 0)

    lane = jnp.arange(L, dtype=jnp.int32)
    lane_base = lane * STRIDE
    ones = jnp.ones((L,), jnp.float32)
    fzero = jnp.zeros((L,), jnp.float32)

    def block_body(bref, c):
        @plsc.parallel_loop(0, BLOCK // L, unroll=8)
        def _(i):
            ib = pl.multiple_of(i * L, L)
            v = bref[pl.ds(ib, L)]
            u = jnp.maximum((v - tmin_eps) * scale, fzero)
            idx0 = u.astype(jnp.int32)
            e = plsc.load_gather(edges_hi_v, [idx0])
            idx = idx0 + (v >= e)
            plsc.addupdate_scatter(h2d, [idx + lane_base], ones)

        return c

    _stream_shard(x_hbm, base, (buf0, buf1), (sem0, sem1), block_body, 0)

    def red_it(j, c):
        jb = pl.multiple_of(j * L, L)
        acc = h2d[pl.ds(jb, L)]
        for l in range(1, L):
            acc = acc + h2d[pl.ds(l * STRIDE + jb, L)]
        h1d[pl.ds(jb, L)] = acc
        return c

    lax.fori_loop(0, NBINS // L, red_it, 0)
    t256 = plsc.load_gather(h2d, [lane_base + NBINS])
    s256 = jnp.sum(t256)
    last = h1d[pl.ds(NBINS - L, L)]
    h1d[pl.ds(NBINS - L, L)] = last + jnp.where(lane == L - 1, s256, 0.0)
    pltpu.sync_copy(
        h1d, out_hbm.at[pl.ds(pl.multiple_of(wid * NBINS, NBINS), NBINS)]
    )


_W1 = float((1 - 0.99))
_W2 = float((1 + 0.99))


@functools.partial(
    pl.kernel,
    out_type=jax.ShapeDtypeStruct((L,), jnp.float32),
    mesh=_mesh,
    compiler_params=_cparams,
    scratch_types=[
        pltpu.VMEM((NW * NBINS,), jnp.float32),
        pltpu.VMEM((EPAD,), jnp.float32),
        pltpu.VMEM((NBINS,), jnp.float32),
        pltpu.VMEM((L,), jnp.float32),
    ],
)
def _epilogue(parts_hbm, edges_hbm, out_hbm, parts_v, edges_v, cum_v, out_v):
    wid = _worker_id()

    @pl.when(wid == 0)
    def _():
        pltpu.sync_copy(parts_hbm, parts_v)
        pltpu.sync_copy(edges_hbm, edges_v)

        def cum_it(j, carry):
            jb = pl.multiple_of(j * L, L)
            acc = parts_v[pl.ds(jb, L)]
            for w in range(1, NW):
                acc = acc + parts_v[pl.ds(w * NBINS + jb, L)]
            c = plsc.cumsum(acc) + carry
            cum_v[pl.ds(jb, L)] = c
            return jnp.max(c)

        total = lax.fori_loop(0, NBINS // L, cum_it, jnp.float32(0.0))

        t1 = total * jnp.float32(_W1) * jnp.float32(0.5)
        t2 = total * jnp.float32(_W2) * jnp.float32(0.5)

        def cnt_it(j, carry):
            lo, hi = carry
            c = cum_v[pl.ds(pl.multiple_of(j * L, L), L)]
            lo = lo + plsc.all_reduce_population_count(c <= t1)
            hi = hi + plsc.all_reduce_population_count(c <= t2)
            return (lo, hi)

        zi = jnp.zeros((L,), jnp.int32)
        lo_idx, hi_idx = lax.fori_loop(0, NBINS // L, cnt_it, (zi, zi))

        minv = plsc.load_gather(edges_v, [lo_idx])
        maxv = plsc.load_gather(edges_v, [hi_idx])
        lane = jnp.arange(L, dtype=jnp.int32)
        out_v[...] = jnp.where(lane == 0, minv, jnp.where(lane == 1, maxv, 0.0))
        pltpu.sync_copy(out_v, out_hbm)


def kernel(tensor):
    parts = _minmax(tensor).reshape(NW, 2, L)
    tmin = parts[:, 0, :].min()
    tmax = parts[:, 1, :].max()
    edges = jnp.linspace(tmin, tmax, NBINS + 1).astype(jnp.float32)
    scale = jnp.float32(256.0) / (tmax - tmin)
    edges_pad = jnp.concatenate([edges, jnp.zeros((EPAD - (NBINS + 1),), jnp.float32)])
    edges_hi_pad = jnp.concatenate([edges[1:], jnp.zeros((EPAD - NBINS,), jnp.float32)])
    tmin_eps = tmin + jnp.float32(0.01) / scale
    params = jnp.concatenate(
        [jnp.stack([tmin, tmax, scale, tmin_eps]), jnp.zeros((L - 4,), jnp.float32)]
    )
    hist_parts = _hist(tensor, edges_hi_pad, params)
    out = _epilogue(hist_parts, edges_pad)
    return (out[0], out[1])
